# baseline (speedup 1.0000x reference)
import functools

import jax
import jax.numpy as jnp
from jax import lax
from jax.experimental import pallas as pl
from jax.experimental.pallas import tpu as pltpu
from jax.experimental.pallas import tpu_sc as plsc

N = 10000
E = 320000
D = 128
T = 100

NC, NS = 2, 16
NW = NC * NS
EPW = E // NW
CW = 80
NCH = EPW // CW
NP = 10240
RPT = NP // NS
NB = 25
RB = N // NB

_MESH = plsc.VectorSubcoreMesh(
    core_axis_name="c", subcore_axis_name="s", num_cores=NC, num_subcores=NS)


@functools.partial(
    pl.kernel,
    out_type=jax.ShapeDtypeStruct((NC * NP, D), jnp.float32),
    mesh=_MESH,
    scratch_types=[
        pltpu.VMEM((NCH, CW), jnp.int32),
        pltpu.VMEM((CW, D), jnp.float32),
        pltpu.VMEM_SHARED((NP, D), jnp.float32),
        pltpu.SemaphoreType.DMA,
    ],
)
def _deg_sc(dst_hbm, ones_hbm, zero_hbm, out_hbm, didx, ones_v, acc_sh, sem):
    c = lax.axis_index("c")
    s = lax.axis_index("s")
    wid = s * NC + c
    pltpu.sync_copy(ones_hbm, ones_v)
    pltpu.sync_copy(dst_hbm.at[wid], didx)

    def zero_stripe(j, _):
        pltpu.sync_copy(zero_hbm, acc_sh.at[pl.ds(s * RPT + j * CW, CW)])
        return 0
    lax.fori_loop(0, RPT // CW, zero_stripe, 0)
    plsc.subcore_barrier()

    def body(i, _):
        pltpu.async_copy(ones_v, acc_sh.at[didx.at[i]], sem, add=True)
        return 0
    lax.fori_loop(0, NCH, body, 0)

    def drain(i, _):
        pltpu.make_async_copy(ones_v, acc_sh.at[didx.at[i]], sem).wait()
        return 0
    lax.fori_loop(0, NCH, drain, 0)
    plsc.subcore_barrier()

    def copy_out(j, _):
        pltpu.sync_copy(acc_sh.at[pl.ds(s * RPT + j * CW, CW)], ones_v)
        pltpu.sync_copy(ones_v,
                        out_hbm.at[pl.ds(c * NP + s * RPT + j * CW, CW)])
        return 0
    lax.fori_loop(0, RPT // CW, copy_out, 0)


@functools.partial(
    pl.kernel,
    out_type=jax.ShapeDtypeStruct((NC * NP, D), jnp.float32),
    mesh=_MESH,
    scratch_types=[
        pltpu.VMEM((CW,), jnp.int32),
        pltpu.VMEM((CW,), jnp.int32),
        pltpu.VMEM((NCH, CW), jnp.int32),
        pltpu.VMEM((CW, D), jnp.float32),
        pltpu.VMEM((CW, D), jnp.float32),
        pltpu.VMEM_SHARED((NP, D), jnp.float32),
        pltpu.SemaphoreType.DMA,
        pltpu.SemaphoreType.DMA,
    ],
)
def _mp_sc(g_hbm, src_hbm, dst_hbm, zero_hbm, out_hbm, sidx_a, sidx_b, didx,
           rows_a, rows_b, acc_sh, sem_a, sem_b):
    c = lax.axis_index("c")
    s = lax.axis_index("s")
    wid = s * NC + c
    ebase = wid * EPW
    pltpu.sync_copy(dst_hbm.at[wid], didx)

    def zero_stripe(j, _):
        pltpu.sync_copy(zero_hbm, acc_sh.at[pl.ds(s * RPT + j * CW, CW)])
        return 0
    lax.fori_loop(0, RPT // CW, zero_stripe, 0)
    plsc.subcore_barrier()

    pltpu.sync_copy(src_hbm.at[pl.ds(ebase, CW)], sidx_a)
    pltpu.async_copy(g_hbm.at[sidx_a], rows_a, sem_a)

    def body(j, _):
        i0 = 2 * j
        pltpu.sync_copy(src_hbm.at[pl.ds(ebase + (i0 + 1) * CW, CW)], sidx_b)
        pltpu.async_copy(g_hbm.at[sidx_b], rows_b, sem_b)
        pltpu.make_async_copy(g_hbm.at[sidx_a], rows_a, sem_a).wait()
        pltpu.sync_copy(rows_a, acc_sh.at[didx.at[i0]], add=True)
        pltpu.sync_copy(src_hbm.at[pl.ds(ebase + (i0 + 2) * CW, CW)], sidx_a)
        pltpu.async_copy(g_hbm.at[sidx_a], rows_a, sem_a)
        pltpu.make_async_copy(g_hbm.at[sidx_b], rows_b, sem_b).wait()
        pltpu.sync_copy(rows_b, acc_sh.at[didx.at[i0 + 1]], add=True)
        return 0
    lax.fori_loop(0, NCH // 2, body, 0)

    pltpu.make_async_copy(g_hbm.at[sidx_a], rows_a, sem_a).wait()
    pltpu.sync_copy(rows_a, acc_sh.at[didx.at[NCH - 1]], add=True)
    plsc.subcore_barrier()

    def copy_out(j, _):
        pltpu.sync_copy(acc_sh.at[pl.ds(s * RPT + j * CW, CW)], rows_a)
        pltpu.sync_copy(rows_a,
                        out_hbm.at[pl.ds(c * NP + s * RPT + j * CW, CW)])
        return 0
    lax.fori_loop(0, RPT // CW, copy_out, 0)


def _matmul_body(x_ref, w_ref, o_ref):
    o_ref[...] = jnp.dot(x_ref[...], w_ref[...],
                         preferred_element_type=jnp.float32)


def _matmul(x, w1):
    return pl.pallas_call(
        _matmul_body,
        grid=(NB,),
        in_specs=[
            pl.BlockSpec((RB, D), lambda i: (i, 0)),
            pl.BlockSpec((D, D), lambda i: (0, 0)),
        ],
        out_specs=pl.BlockSpec((RB, D), lambda i: (i, 0)),
        out_shape=jax.ShapeDtypeStruct((N, D), jnp.float32),
    )(x, w1)


def _scale_body(deg_ref, m_ref, g_ref, dinv_ref):
    deg = deg_ref[0, :, 0:1] + deg_ref[1, :, 0:1] + 1.0
    dinv = lax.rsqrt(deg)
    dinv_ref[...] = dinv
    g_ref[...] = m_ref[...] * dinv


def _scale(deg2, m1):
    return pl.pallas_call(
        _scale_body,
        grid=(NB,),
        in_specs=[
            pl.BlockSpec((NC, RB, D), lambda i: (0, i, 0)),
            pl.BlockSpec((RB, D), lambda i: (i, 0)),
        ],
        out_specs=[
            pl.BlockSpec((RB, D), lambda i: (i, 0)),
            pl.BlockSpec((RB, 1), lambda i: (i, 0)),
        ],
        out_shape=[
            jax.ShapeDtypeStruct((N, D), jnp.float32),
            jax.ShapeDtypeStruct((N, 1), jnp.float32),
        ],
    )(deg2, m1)


def _mat2_body(acc_ref, g1_ref, dinv_ref, b1_ref, w2_ref, g2_ref):
    dinv = dinv_ref[...]
    h1 = jax.nn.relu(dinv * (acc_ref[0] + acc_ref[1] + g1_ref[...])
                     + b1_ref[...])
    g2_ref[...] = jnp.dot(h1, w2_ref[...],
                          preferred_element_type=jnp.float32) * dinv


def _mat2(acc1, g1, dinv, b1, w2):
    return pl.pallas_call(
        _mat2_body,
        grid=(NB,),
        in_specs=[
            pl.BlockSpec((NC, RB, D), lambda i: (0, i, 0)),
            pl.BlockSpec((RB, D), lambda i: (i, 0)),
            pl.BlockSpec((RB, 1), lambda i: (i, 0)),
            pl.BlockSpec((1, D), lambda i: (0, 0)),
            pl.BlockSpec((D, D), lambda i: (0, 0)),
        ],
        out_specs=pl.BlockSpec((RB, D), lambda i: (i, 0)),
        out_shape=jax.ShapeDtypeStruct((N, D), jnp.float32),
    )(acc1, g1, dinv, b1, w2)


def _final_body(acc_ref, g2_ref, dinv_ref, b2_ref, tid_ref,
                wout_ref, bout_ref, wenc_ref, benc_ref, wattn_ref, battn_ref,
                wcls_ref, bcls_ref,
                logits_ref, attn_ref, embs_ref, pool_acc, cnt_acc):
    i = pl.program_id(0)
    nb = pl.num_programs(0)

    @pl.when(i == 0)
    def _init():
        pool_acc[...] = jnp.zeros_like(pool_acc)
        cnt_acc[...] = jnp.zeros_like(cnt_acc)

    h2 = jax.nn.relu(dinv_ref[...] * (acc_ref[0] + acc_ref[1] + g2_ref[...])
                     + b2_ref[...])
    tids = tid_ref[0, 0:1, :]
    iota = lax.broadcasted_iota(jnp.int32, (128, tids.shape[1]), 0)
    oh = (tids == iota).astype(jnp.float32)
    pool_acc[...] += jnp.dot(oh, h2, preferred_element_type=jnp.float32)
    cnt_acc[...] += jnp.sum(oh, axis=1, keepdims=True)

    @pl.when(i == nb - 1)
    def _fin():
        pooled = pool_acc[...] / jnp.maximum(cnt_acc[...], 1.0)
        embs = jnp.dot(pooled, wout_ref[...],
                       preferred_element_type=jnp.float32) + bout_ref[...]
        z = jax.nn.relu(jnp.dot(embs, wenc_ref[...],
                                preferred_element_type=jnp.float32)
                        + benc_ref[...])
        sc = jnp.dot(z, wattn_ref[...],
                     preferred_element_type=jnp.float32) + battn_ref[...]
        rid = lax.broadcasted_iota(jnp.int32, (128, 1), 0)
        sc = jnp.where(rid < T, sc, -1e30)
        m = jnp.max(sc, axis=0, keepdims=True)
        e = jnp.where(rid < T, jnp.exp(sc - m), 0.0)
        attn = e / jnp.sum(e, axis=0, keepdims=True)
        slide = jnp.sum(attn * z, axis=0, keepdims=True)
        logits_ref[...] = jnp.dot(slide, wcls_ref[...],
                                  preferred_element_type=jnp.float32) \
            + bcls_ref[...]
        attn_ref[...] = attn[:T, :]
        embs_ref[...] = embs[:T, :]


def _final(acc2, g2, dinv, b2, tid3, wout, bout, wenc, benc, wattn, battn,
           wcls, bcls):
    full = lambda shape: pl.BlockSpec(shape, lambda i: tuple(0 for _ in shape))
    return pl.pallas_call(
        _final_body,
        grid=(NB,),
        in_specs=[
            pl.BlockSpec((NC, RB, D), lambda i: (0, i, 0)),
            pl.BlockSpec((RB, D), lambda i: (i, 0)),
            pl.BlockSpec((RB, 1), lambda i: (i, 0)),
            full((1, D)),
            pl.BlockSpec((1, 1, RB), lambda i: (i, 0, 0)),
            full((D, D)),
            full((1, D)),
            full((D, D)),
            full((1, D)),
            full((D, 1)),
            full((1, 1)),
            full((D, 2)),
            full((1, 2)),
        ],
        out_specs=[
            full((1, 2)),
            full((T, 1)),
            full((T, D)),
        ],
        out_shape=[
            jax.ShapeDtypeStruct((1, 2), jnp.float32),
            jax.ShapeDtypeStruct((T, 1), jnp.float32),
            jax.ShapeDtypeStruct((T, D), jnp.float32),
        ],
        scratch_shapes=[
            pltpu.VMEM((128, D), jnp.float32),
            pltpu.VMEM((128, 1), jnp.float32),
        ],
    )(acc2, g2, dinv, b2, tid3, wout, bout, wenc, benc, wattn, battn,
      wcls, bcls)


def kernel(x, edge_index, tile_ids, W1, b1, W2, b2, Wout, bout, Wenc, benc,
           Wattn, battn, Wcls, bcls):
    src = edge_index[0]
    dst = edge_index[1].reshape(NW, NCH, CW)
    zero128 = jnp.zeros((CW, D), jnp.float32)
    ones128 = jnp.ones((CW, D), jnp.float32)
    tid3 = tile_ids.astype(jnp.int32).reshape(NB, 1, RB)

    m1 = _matmul(x, W1)
    deg2 = _deg_sc(dst, ones128, zero128).reshape(NC, NP, D)
    g1, dinv = _scale(deg2, m1)
    acc1 = _mp_sc(g1, src, dst, zero128).reshape(NC, NP, D)
    g2 = _mat2(acc1, g1, dinv, b1.reshape(1, D), W2)
    acc2 = _mp_sc(g2, src, dst, zero128).reshape(NC, NP, D)
    logits, attn, embs = _final(
        acc2, g2, dinv, b2.reshape(1, D), tid3,
        Wout, bout.reshape(1, D), Wenc, benc.reshape(1, D),
        Wattn, battn.reshape(1, 1), Wcls, bcls.reshape(1, 2))
    return (logits, attn, embs)


# speedup vs baseline: 1.0046x; 1.0046x over previous
import functools

import jax
import jax.numpy as jnp
from jax import lax
from jax.experimental import pallas as pl
from jax.experimental.pallas import tpu as pltpu
from jax.experimental.pallas import tpu_sc as plsc

N = 10000
E = 320000
D = 128
T = 100

NC, NS = 2, 16
NW = NC * NS
EPW = E // NW
CW = 80
NCH = EPW // CW
NP = 10240
RPT = NP // NS
NB = 25
RB = N // NB

_MESH = plsc.VectorSubcoreMesh(
    core_axis_name="c", subcore_axis_name="s", num_cores=NC, num_subcores=NS)


@functools.partial(
    pl.kernel,
    out_type=jax.ShapeDtypeStruct((NC * NP, D), jnp.float32),
    mesh=_MESH,
    scratch_types=[
        pltpu.VMEM((NCH, CW), jnp.int32),
        pltpu.VMEM((CW, D), jnp.float32),
        pltpu.VMEM_SHARED((NP, D), jnp.float32),
        pltpu.SemaphoreType.DMA,
    ],
)
def _deg_sc(dst_hbm, ones_hbm, zero_hbm, out_hbm, didx, ones_v, acc_sh, sem):
    c = lax.axis_index("c")
    s = lax.axis_index("s")
    wid = s * NC + c
    pltpu.sync_copy(ones_hbm, ones_v)
    pltpu.sync_copy(dst_hbm.at[wid], didx)

    def zero_stripe(j, _):
        pltpu.sync_copy(zero_hbm, acc_sh.at[pl.ds(s * RPT + j * CW, CW)])
        return 0
    lax.fori_loop(0, RPT // CW, zero_stripe, 0)
    plsc.subcore_barrier()

    def body(i, _):
        pltpu.async_copy(ones_v, acc_sh.at[didx.at[i]], sem, add=True)
        return 0
    lax.fori_loop(0, NCH, body, 0)

    def drain(i, _):
        pltpu.make_async_copy(ones_v, acc_sh.at[didx.at[i]], sem).wait()
        return 0
    lax.fori_loop(0, NCH, drain, 0)
    plsc.subcore_barrier()

    def copy_out(j, _):
        pltpu.sync_copy(acc_sh.at[pl.ds(s * RPT + j * CW, CW)], ones_v)
        pltpu.sync_copy(ones_v,
                        out_hbm.at[pl.ds(c * NP + s * RPT + j * CW, CW)])
        return 0
    lax.fori_loop(0, RPT // CW, copy_out, 0)


@functools.partial(
    pl.kernel,
    out_type=jax.ShapeDtypeStruct((NC * NP, D), jnp.float32),
    mesh=_MESH,
    scratch_types=[
        pltpu.VMEM((CW,), jnp.int32),
        pltpu.VMEM((CW,), jnp.int32),
        pltpu.VMEM((NCH, CW), jnp.int32),
        pltpu.VMEM((CW, D), jnp.float32),
        pltpu.VMEM((CW, D), jnp.float32),
        pltpu.VMEM_SHARED((NP, D), jnp.float32),
        pltpu.SemaphoreType.DMA,
        pltpu.SemaphoreType.DMA,
    ],
)
def _mp_sc(g_hbm, src_hbm, dst_hbm, zero_hbm, out_hbm, sidx_a, sidx_b, didx,
           rows_a, rows_b, acc_sh, sem_a, sem_b):
    c = lax.axis_index("c")
    s = lax.axis_index("s")
    wid = s * NC + c
    ebase = wid * EPW
    pltpu.sync_copy(dst_hbm.at[wid], didx)

    def zero_stripe(j, _):
        pltpu.sync_copy(zero_hbm, acc_sh.at[pl.ds(s * RPT + j * CW, CW)])
        return 0
    lax.fori_loop(0, RPT // CW, zero_stripe, 0)
    plsc.subcore_barrier()

    pltpu.sync_copy(src_hbm.at[pl.ds(ebase, CW)], sidx_a)
    pltpu.async_copy(g_hbm.at[sidx_a], rows_a, sem_a)

    def body(j, _):
        i0 = 2 * j
        pltpu.sync_copy(src_hbm.at[pl.ds(ebase + (i0 + 1) * CW, CW)], sidx_b)
        pltpu.async_copy(g_hbm.at[sidx_b], rows_b, sem_b)
        pltpu.make_async_copy(g_hbm.at[sidx_a], rows_a, sem_a).wait()
        pltpu.sync_copy(rows_a, acc_sh.at[didx.at[i0]], add=True)
        pltpu.sync_copy(src_hbm.at[pl.ds(ebase + (i0 + 2) * CW, CW)], sidx_a)
        pltpu.async_copy(g_hbm.at[sidx_a], rows_a, sem_a)
        pltpu.make_async_copy(g_hbm.at[sidx_b], rows_b, sem_b).wait()
        pltpu.sync_copy(rows_b, acc_sh.at[didx.at[i0 + 1]], add=True)
        return 0
    lax.fori_loop(0, NCH // 2, body, 0)

    pltpu.make_async_copy(g_hbm.at[sidx_a], rows_a, sem_a).wait()
    pltpu.sync_copy(rows_a, acc_sh.at[didx.at[NCH - 1]], add=True)
    plsc.subcore_barrier()

    def copy_out(j, _):
        pltpu.sync_copy(acc_sh.at[pl.ds(s * RPT + j * CW, CW)], rows_a)
        pltpu.sync_copy(rows_a,
                        out_hbm.at[pl.ds(c * NP + s * RPT + j * CW, CW)])
        return 0
    lax.fori_loop(0, RPT // CW, copy_out, 0)


def _mat1_body(deg_ref, x_ref, w_ref, g_ref, dinv_ref):
    deg = deg_ref[0, :, 0:1] + deg_ref[1, :, 0:1] + 1.0
    dinv = lax.rsqrt(deg)
    dinv_ref[...] = dinv
    g_ref[...] = jnp.dot(x_ref[...], w_ref[...],
                         preferred_element_type=jnp.float32) * dinv


def _mat1(deg2, x, w1):
    return pl.pallas_call(
        _mat1_body,
        grid=(NB,),
        in_specs=[
            pl.BlockSpec((NC, RB, D), lambda i: (0, i, 0)),
            pl.BlockSpec((RB, D), lambda i: (i, 0)),
            pl.BlockSpec((D, D), lambda i: (0, 0)),
        ],
        out_specs=[
            pl.BlockSpec((RB, D), lambda i: (i, 0)),
            pl.BlockSpec((RB, 1), lambda i: (i, 0)),
        ],
        out_shape=[
            jax.ShapeDtypeStruct((N, D), jnp.float32),
            jax.ShapeDtypeStruct((N, 1), jnp.float32),
        ],
    )(deg2, x, w1)


def _mat2_body(acc_ref, g1_ref, dinv_ref, b1_ref, w2_ref, g2_ref):
    dinv = dinv_ref[...]
    h1 = jax.nn.relu(dinv * (acc_ref[0] + acc_ref[1] + g1_ref[...])
                     + b1_ref[...])
    g2_ref[...] = jnp.dot(h1, w2_ref[...],
                          preferred_element_type=jnp.float32) * dinv


def _mat2(acc1, g1, dinv, b1, w2):
    return pl.pallas_call(
        _mat2_body,
        grid=(NB,),
        in_specs=[
            pl.BlockSpec((NC, RB, D), lambda i: (0, i, 0)),
            pl.BlockSpec((RB, D), lambda i: (i, 0)),
            pl.BlockSpec((RB, 1), lambda i: (i, 0)),
            pl.BlockSpec((1, D), lambda i: (0, 0)),
            pl.BlockSpec((D, D), lambda i: (0, 0)),
        ],
        out_specs=pl.BlockSpec((RB, D), lambda i: (i, 0)),
        out_shape=jax.ShapeDtypeStruct((N, D), jnp.float32),
    )(acc1, g1, dinv, b1, w2)


def _final_body(acc_ref, g2_ref, dinv_ref, b2_ref, tid_ref,
                wout_ref, bout_ref, wenc_ref, benc_ref, wattn_ref, battn_ref,
                wcls_ref, bcls_ref,
                logits_ref, attn_ref, embs_ref, pool_acc, cnt_acc):
    i = pl.program_id(0)
    nb = pl.num_programs(0)

    @pl.when(i == 0)
    def _init():
        pool_acc[...] = jnp.zeros_like(pool_acc)
        cnt_acc[...] = jnp.zeros_like(cnt_acc)

    h2 = jax.nn.relu(dinv_ref[...] * (acc_ref[0] + acc_ref[1] + g2_ref[...])
                     + b2_ref[...])
    tids = tid_ref[0, 0:1, :]
    iota = lax.broadcasted_iota(jnp.int32, (128, tids.shape[1]), 0)
    oh = (tids == iota).astype(jnp.float32)
    pool_acc[...] += jnp.dot(oh, h2, preferred_element_type=jnp.float32)
    cnt_acc[...] += jnp.sum(oh, axis=1, keepdims=True)

    @pl.when(i == nb - 1)
    def _fin():
        pooled = pool_acc[...] / jnp.maximum(cnt_acc[...], 1.0)
        embs = jnp.dot(pooled, wout_ref[...],
                       preferred_element_type=jnp.float32) + bout_ref[...]
        z = jax.nn.relu(jnp.dot(embs, wenc_ref[...],
                                preferred_element_type=jnp.float32)
                        + benc_ref[...])
        sc = jnp.dot(z, wattn_ref[...],
                     preferred_element_type=jnp.float32) + battn_ref[...]
        rid = lax.broadcasted_iota(jnp.int32, (128, 1), 0)
        sc = jnp.where(rid < T, sc, -1e30)
        m = jnp.max(sc, axis=0, keepdims=True)
        e = jnp.where(rid < T, jnp.exp(sc - m), 0.0)
        attn = e / jnp.sum(e, axis=0, keepdims=True)
        slide = jnp.sum(attn * z, axis=0, keepdims=True)
        logits_ref[...] = jnp.dot(slide, wcls_ref[...],
                                  preferred_element_type=jnp.float32) \
            + bcls_ref[...]
        attn_ref[...] = attn[:T, :]
        embs_ref[...] = embs[:T, :]


def _final(acc2, g2, dinv, b2, tid3, wout, bout, wenc, benc, wattn, battn,
           wcls, bcls):
    full = lambda shape: pl.BlockSpec(shape, lambda i: tuple(0 for _ in shape))
    return pl.pallas_call(
        _final_body,
        grid=(NB,),
        in_specs=[
            pl.BlockSpec((NC, RB, D), lambda i: (0, i, 0)),
            pl.BlockSpec((RB, D), lambda i: (i, 0)),
            pl.BlockSpec((RB, 1), lambda i: (i, 0)),
            full((1, D)),
            pl.BlockSpec((1, 1, RB), lambda i: (i, 0, 0)),
            full((D, D)),
            full((1, D)),
            full((D, D)),
            full((1, D)),
            full((D, 1)),
            full((1, 1)),
            full((D, 2)),
            full((1, 2)),
        ],
        out_specs=[
            full((1, 2)),
            full((T, 1)),
            full((T, D)),
        ],
        out_shape=[
            jax.ShapeDtypeStruct((1, 2), jnp.float32),
            jax.ShapeDtypeStruct((T, 1), jnp.float32),
            jax.ShapeDtypeStruct((T, D), jnp.float32),
        ],
        scratch_shapes=[
            pltpu.VMEM((128, D), jnp.float32),
            pltpu.VMEM((128, 1), jnp.float32),
        ],
    )(acc2, g2, dinv, b2, tid3, wout, bout, wenc, benc, wattn, battn,
      wcls, bcls)


def kernel(x, edge_index, tile_ids, W1, b1, W2, b2, Wout, bout, Wenc, benc,
           Wattn, battn, Wcls, bcls):
    src = edge_index[0]
    dst = edge_index[1].reshape(NW, NCH, CW)
    zero128 = jnp.zeros((CW, D), jnp.float32)
    ones128 = jnp.ones((CW, D), jnp.float32)
    tid3 = tile_ids.astype(jnp.int32).reshape(NB, 1, RB)

    deg2 = _deg_sc(dst, ones128, zero128).reshape(NC, NP, D)
    g1, dinv = _mat1(deg2, x, W1)
    acc1 = _mp_sc(g1, src, dst, zero128).reshape(NC, NP, D)
    g2 = _mat2(acc1, g1, dinv, b1.reshape(1, D), W2)
    acc2 = _mp_sc(g2, src, dst, zero128).reshape(NC, NP, D)
    logits, attn, embs = _final(
        acc2, g2, dinv, b2.reshape(1, D), tid3,
        Wout, bout.reshape(1, D), Wenc, benc.reshape(1, D),
        Wattn, battn.reshape(1, 1), Wcls, bcls.reshape(1, 2))
    return (logits, attn, embs)
